# baseline (speedup 1.0000x reference)
import jax
import jax.numpy as jnp
from jax.experimental import pallas as pl
from jax.experimental.pallas import tpu as pltpu

_N = 4096
_F_IN = 128
_SUPPORT = 3
_F_OUT = 128
_BM = 512


def _gcn_block(b_ref, x_ref, w_ref, bias_ref, o_ref, y_ref):
    @pl.when(pl.program_id(0) == 0)
    def _():
        x = x_ref[...].astype(jnp.bfloat16)
        for i in range(_SUPPORT):
            w_i = w_ref[i * _F_IN:(i + 1) * _F_IN, :].astype(jnp.bfloat16)
            y_ref[i, :, :] = jnp.dot(
                x, w_i, preferred_element_type=jnp.float32
            ).astype(jnp.bfloat16)

    acc = jnp.zeros(o_ref.shape, jnp.float32)
    for i in range(_SUPPORT):
        acc += jnp.dot(
            b_ref[i].astype(jnp.bfloat16),
            y_ref[i],
            preferred_element_type=jnp.float32,
        )
    o_ref[...] = acc + bias_ref[...].astype(jnp.float32)


def kernel(features, basis, weight, bias):
    bias2 = bias.reshape(1, _F_OUT)
    return pl.pallas_call(
        _gcn_block,
        grid=(_N // _BM,),
        in_specs=[
            pl.BlockSpec((_SUPPORT, _BM, _N), lambda m: (0, m, 0)),
            pl.BlockSpec((_N, _F_IN), lambda m: (0, 0)),
            pl.BlockSpec((_F_IN * _SUPPORT, _F_OUT), lambda m: (0, 0)),
            pl.BlockSpec((1, _F_OUT), lambda m: (0, 0)),
        ],
        out_specs=pl.BlockSpec((_BM, _F_OUT), lambda m: (m, 0)),
        out_shape=jax.ShapeDtypeStruct((_N, _F_OUT), jnp.float32),
        scratch_shapes=[pltpu.VMEM((_SUPPORT, _N, _F_OUT), jnp.bfloat16)],
        compiler_params=pltpu.CompilerParams(
            dimension_semantics=("arbitrary",)
        ),
    )(basis, features, weight, bias2)


# speedup vs baseline: 1.0449x; 1.0449x over previous
import jax
import jax.numpy as jnp
from jax.experimental import pallas as pl
from jax.experimental.pallas import tpu as pltpu

_N = 4096
_F_IN = 128
_SUPPORT = 3
_F_OUT = 128
_BM = 256


def _gcn_block(b_ref, x_ref, w_ref, bias_ref, o_ref, y_ref):
    @pl.when(pl.program_id(0) == 0)
    def _():
        x = x_ref[...].astype(jnp.bfloat16)
        for i in range(_SUPPORT):
            w_i = w_ref[i * _F_IN:(i + 1) * _F_IN, :].astype(jnp.bfloat16)
            y_ref[i, :, :] = jnp.dot(
                x, w_i, preferred_element_type=jnp.float32
            ).astype(jnp.bfloat16)

    acc = jnp.zeros(o_ref.shape, jnp.float32)
    for i in range(_SUPPORT):
        acc += jnp.dot(
            b_ref[i].astype(jnp.bfloat16),
            y_ref[i],
            preferred_element_type=jnp.float32,
        )
    o_ref[...] = acc + bias_ref[...].astype(jnp.float32)


def kernel(features, basis, weight, bias):
    bias2 = bias.reshape(1, _F_OUT)
    return pl.pallas_call(
        _gcn_block,
        grid=(_N // _BM,),
        in_specs=[
            pl.BlockSpec((_SUPPORT, _BM, _N), lambda m: (0, m, 0)),
            pl.BlockSpec((_N, _F_IN), lambda m: (0, 0)),
            pl.BlockSpec((_F_IN * _SUPPORT, _F_OUT), lambda m: (0, 0)),
            pl.BlockSpec((1, _F_OUT), lambda m: (0, 0)),
        ],
        out_specs=pl.BlockSpec((_BM, _F_OUT), lambda m: (m, 0)),
        out_shape=jax.ShapeDtypeStruct((_N, _F_OUT), jnp.float32),
        scratch_shapes=[pltpu.VMEM((_SUPPORT, _N, _F_OUT), jnp.bfloat16)],
        compiler_params=pltpu.CompilerParams(
            dimension_semantics=("arbitrary",)
        ),
    )(basis, features, weight, bias2)


# speedup vs baseline: 1.0499x; 1.0048x over previous
import jax
import jax.numpy as jnp
from jax.experimental import pallas as pl
from jax.experimental.pallas import tpu as pltpu

_N = 4096
_F_IN = 128
_SUPPORT = 3
_F_OUT = 128
_BM = 512


def _gcn_block(b_ref, x_ref, w_ref, bias_ref, o_ref, y_ref):
    i = pl.program_id(1)

    @pl.when((pl.program_id(0) == 0) & (i == 0))
    def _():
        x = x_ref[...].astype(jnp.bfloat16)
        for s in range(_SUPPORT):
            w_s = w_ref[s * _F_IN:(s + 1) * _F_IN, :].astype(jnp.bfloat16)
            y_ref[s, :, :] = jnp.dot(
                x, w_s, preferred_element_type=jnp.float32
            ).astype(jnp.bfloat16)

    contrib = jnp.dot(
        b_ref[...].astype(jnp.bfloat16),
        y_ref[i],
        preferred_element_type=jnp.float32,
    )

    @pl.when(i == 0)
    def _():
        o_ref[...] = contrib + bias_ref[...].astype(jnp.float32)

    @pl.when(i != 0)
    def _():
        o_ref[...] += contrib


def kernel(features, basis, weight, bias):
    bias2 = bias.reshape(1, _F_OUT)
    basis2 = basis.reshape(_SUPPORT * _N, _N)
    nb = _N // _BM
    return pl.pallas_call(
        _gcn_block,
        grid=(nb, _SUPPORT),
        in_specs=[
            pl.BlockSpec((_BM, _N), lambda m, i: (i * nb + m, 0)),
            pl.BlockSpec((_N, _F_IN), lambda m, i: (0, 0)),
            pl.BlockSpec((_F_IN * _SUPPORT, _F_OUT), lambda m, i: (0, 0)),
            pl.BlockSpec((1, _F_OUT), lambda m, i: (0, 0)),
        ],
        out_specs=pl.BlockSpec((_BM, _F_OUT), lambda m, i: (m, 0)),
        out_shape=jax.ShapeDtypeStruct((_N, _F_OUT), jnp.float32),
        scratch_shapes=[pltpu.VMEM((_SUPPORT, _N, _F_OUT), jnp.bfloat16)],
        compiler_params=pltpu.CompilerParams(
            dimension_semantics=("arbitrary", "arbitrary")
        ),
    )(basis2, features, weight, bias2)
